# baseline (speedup 1.0000x reference)
import functools

import jax
import jax.numpy as jnp
from jax import lax
from jax.experimental import pallas as pl
from jax.experimental.pallas import tpu as pltpu
from jax.experimental.pallas import tpu_sc as plsc

SCALE = 10.0
NUM_ROWS = 16384
NUM_COLS = 26
EMBED_DIM = 32
VOCAB = 1000000
NW = 32

TCH = 800
NCHT = VOCAB // TCH
ITER1 = (NCHT + NW - 1) // NW

ROWS_W = NUM_ROWS // NW
PER_W = ROWS_W * NUM_COLS
NBLK = 4 * NUM_COLS
NB = 4
NSB = 4
OUT_WORDS = NUM_ROWS * NUM_COLS * EMBED_DIM

_mesh = plsc.VectorSubcoreMesh(core_axis_name="c", subcore_axis_name="s")


@functools.partial(
    pl.kernel,
    out_type=jax.ShapeDtypeStruct((VOCAB, EMBED_DIM), jnp.float32),
    mesh=_mesh,
    scratch_types=[
        pltpu.VMEM((2, EMBED_DIM, TCH), jnp.float32),
        pltpu.VMEM((2, TCH, EMBED_DIM), jnp.float32),
        pltpu.SemaphoreType.DMA,
        pltpu.SemaphoreType.DMA,
    ],
    compiler_params=pltpu.CompilerParams(use_tc_tiling_on_sc=False, needs_layout_passes=False),
)
def _transpose_table(tt_hbm, out_hbm, tbuf, obuf, lsem, ssem):
    wid = lax.axis_index("s") * 2 + lax.axis_index("c")
    lane = lax.iota(jnp.int32, 16)

    pltpu.async_copy(tt_hbm.at[:, pl.ds(wid * TCH, TCH)], tbuf.at[0], lsem)

    def step(n, carry):
        b = lax.rem(n, 2)
        g = n * NW + wid
        col0 = g * TCH

        @pl.when(g + NW < NCHT)
        def _next_load():
            pltpu.async_copy(
                tt_hbm.at[:, pl.ds((g + NW) * TCH, TCH)], tbuf.at[1 - b], lsem
            )

        @pl.when(g < NCHT)
        def _work():
            pltpu.make_async_copy(
                tt_hbm.at[:, pl.ds(col0, TCH)], tbuf.at[b], lsem
            ).wait()

            @pl.when(n >= 2)
            def _wait_store():
                pltpu.make_async_copy(
                    obuf.at[b], out_hbm.at[pl.ds(0, TCH)], ssem
                ).wait()

            @plsc.parallel_loop(0, TCH // 16, unroll=4)
            def _t(m):
                rows = m * 16 + lane
                for f in range(EMBED_DIM):
                    vals = tbuf[b, f, pl.ds(m * 16, 16)]
                    plsc.store_scatter(obuf, [jnp.full((16,), b, jnp.int32),
                                              rows,
                                              jnp.full((16,), f, jnp.int32)],
                                       vals)

            pltpu.async_copy(obuf.at[b], out_hbm.at[pl.ds(col0, TCH)], ssem)

        return carry

    lax.fori_loop(0, ITER1, step, 0)

    for _ in range(2):
        pltpu.make_async_copy(
            obuf.at[0], out_hbm.at[pl.ds(0, TCH)], ssem
        ).wait()


@functools.partial(
    pl.kernel,
    out_type=jax.ShapeDtypeStruct((OUT_WORDS,), jnp.float32),
    mesh=_mesh,
    scratch_types=[
        pltpu.VMEM((PER_W,), jnp.int32),
        pltpu.VMEM((NB, 128), jnp.int32),
        pltpu.VMEM((NB, 128, EMBED_DIM), jnp.float32),
        pltpu.VMEM((NSB, 4 * 1024), jnp.float32),
        pltpu.SemaphoreType.DMA,
        pltpu.SemaphoreType.DMA,
    ],
    compiler_params=pltpu.CompilerParams(use_tc_tiling_on_sc=False, needs_layout_passes=False),
)
def _gather_scale(x_hbm, t_hbm, out_hbm, idx_raw, bidx, gbuf, obuf, gsem, ssem):
    wid = lax.axis_index("s") * 2 + lax.axis_index("c")
    lane = lax.iota(jnp.int32, 16)
    lane26 = lane * NUM_COLS

    pltpu.sync_copy(x_hbm.at[pl.ds(wid * PER_W, PER_W)], idx_raw)

    def build_and_issue(n, slot):
        c = lax.shift_right_logical(n, 2)
        rb = lax.bitwise_and(n, 3)

        @plsc.parallel_loop(0, 8, unroll=8)
        def _b(t):
            base = (rb * 128 + t * 16) * NUM_COLS + c
            vals = plsc.load_gather(idx_raw, [lane26 + base])
            bidx[slot, pl.ds(t * 16, 16)] = vals

        pltpu.async_copy(t_hbm.at[bidx.at[slot]], gbuf.at[slot], gsem)

    for p in range(NB - 1):
        build_and_issue(jnp.int32(p), p)

    def step(n, carry):
        slot = lax.rem(n, NB)
        pslot = lax.rem(n + NB - 1, NB)
        sb = lax.rem(n, NSB)
        c = lax.shift_right_logical(n, 2)
        rb = lax.bitwise_and(n, 3)
        rbg = wid * 4 + rb

        @pl.when(n + NB - 1 < NBLK)
        def _lookahead():
            build_and_issue(n + NB - 1, pslot)

        @pl.when(n >= NSB)
        def _wait_store():
            pltpu.make_async_copy(
                obuf.at[sb], out_hbm.at[pl.ds(0, 4 * 1024)], ssem
            ).wait()

        pltpu.make_async_copy(
            t_hbm.at[bidx.at[slot]], gbuf.at[slot], gsem
        ).wait()

        @plsc.parallel_loop(0, EMBED_DIM, unroll=4)
        def _t(f):
            for m in range(8):
                rows = m * 16 + lane
                vals = plsc.load_gather(
                    gbuf,
                    [jnp.full((16,), slot, jnp.int32),
                     rows,
                     jnp.full((16,), f, jnp.int32)],
                )
                obuf[sb, pl.ds(f * 128 + m * 16, 16)] = vals * SCALE

        for fb in range(4):
            base = ((c * 4 + fb) * 128 + rbg) * 1024
            pltpu.async_copy(
                obuf.at[sb, pl.ds(fb * 1024, 1024)],
                out_hbm.at[pl.ds(base, 1024)],
                ssem,
            )
        return carry

    lax.fori_loop(0, NBLK, step, 0)

    for _ in range(NSB):
        pltpu.make_async_copy(
            obuf.at[0], out_hbm.at[pl.ds(0, 4 * 1024)], ssem
        ).wait()


@jax.jit
def kernel(x, table):
    tlin = _transpose_table(table.T)
    oflat = _gather_scale(x.reshape(-1), tlin)
    out = oflat.reshape(NUM_COLS, 4, 128, 8, 128)
    return out.transpose(2, 4, 0, 1, 3).reshape(NUM_ROWS, NUM_COLS, EMBED_DIM)


# speedup vs baseline: 4.3008x; 4.3008x over previous
import functools

import jax
import jax.numpy as jnp
from jax import lax
from jax.experimental import pallas as pl
from jax.experimental.pallas import tpu as pltpu
from jax.experimental.pallas import tpu_sc as plsc

SCALE = 10.0
NUM_ROWS = 16384
NUM_COLS = 26
EMBED_DIM = 32
VOCAB = 1000000
NW = 32

TCH = 512
VMAIN = (VOCAB // TCH) * TCH
NCHT = VMAIN // TCH
ITER1 = (NCHT + NW - 1) // NW
TAIL = VOCAB - VMAIN

ROWS_W = NUM_ROWS // NW
PER_W = ROWS_W * NUM_COLS
NBLK = 4 * NUM_COLS
NB = 4
NSB = 4
OUT_WORDS = NUM_ROWS * NUM_COLS * EMBED_DIM

_mesh = plsc.VectorSubcoreMesh(core_axis_name="c", subcore_axis_name="s")


@functools.partial(
    pl.kernel,
    out_type=jax.ShapeDtypeStruct((VOCAB * EMBED_DIM,), jnp.float32),
    mesh=_mesh,
    scratch_types=[
        pltpu.VMEM((2, EMBED_DIM, TCH), jnp.float32),
        pltpu.VMEM((2, TCH * EMBED_DIM), jnp.float32),
        pltpu.VMEM((TAIL * EMBED_DIM,), jnp.float32),
        pltpu.SemaphoreType.DMA,
        pltpu.SemaphoreType.DMA,
    ],
    compiler_params=pltpu.CompilerParams(
        use_tc_tiling_on_sc=True, needs_layout_passes=False
    ),
)
def _transpose_table(tt_hbm, tail_hbm, out_hbm, tbuf, obuf, tailv, lsem, ssem):
    wid = lax.axis_index("s") * 2 + lax.axis_index("c")
    lane = lax.iota(jnp.int32, 16)
    lane32 = lane * EMBED_DIM

    @pl.when(wid == 0)
    def _tail():
        pltpu.sync_copy(tail_hbm, tailv)
        pltpu.sync_copy(tailv, out_hbm.at[pl.ds(VMAIN * EMBED_DIM, TAIL * EMBED_DIM)])

    pltpu.async_copy(tt_hbm.at[:, pl.ds(wid * TCH, TCH)], tbuf.at[0], lsem)

    def step(n, carry):
        b = lax.rem(n, 2)
        g = n * NW + wid

        @pl.when(g + NW < NCHT)
        def _next_load():
            pltpu.async_copy(
                tt_hbm.at[:, pl.ds((g + NW) * TCH, TCH)], tbuf.at[1 - b], lsem
            )

        @pl.when(g < NCHT)
        def _work():
            pltpu.make_async_copy(
                tt_hbm.at[:, pl.ds(g * TCH, TCH)], tbuf.at[b], lsem
            ).wait()

            @pl.when(n >= 2)
            def _wait_store():
                pltpu.make_async_copy(
                    obuf.at[b], out_hbm.at[pl.ds(0, TCH * EMBED_DIM)], ssem
                ).wait()

            @plsc.parallel_loop(0, TCH // 16, unroll=4)
            def _t(m):
                dst0 = lane32 + m * 16 * EMBED_DIM
                for f in range(EMBED_DIM):
                    vals = tbuf[b, f, pl.ds(m * 16, 16)]
                    plsc.store_scatter(
                        obuf,
                        [jnp.full((16,), b, jnp.int32), dst0 + f],
                        vals,
                    )

            pltpu.async_copy(
                obuf.at[b],
                out_hbm.at[pl.ds(g * TCH * EMBED_DIM, TCH * EMBED_DIM)],
                ssem,
            )

        return carry

    lax.fori_loop(0, ITER1, step, 0)

    for _ in range(2):
        pltpu.make_async_copy(
            obuf.at[0], out_hbm.at[pl.ds(0, TCH * EMBED_DIM)], ssem
        ).wait()


@functools.partial(
    pl.kernel,
    out_type=jax.ShapeDtypeStruct((OUT_WORDS,), jnp.float32),
    mesh=_mesh,
    scratch_types=[
        pltpu.VMEM((PER_W,), jnp.int32),
        pltpu.VMEM((NB, 128), jnp.int32),
        pltpu.VMEM((NB, 128, EMBED_DIM), jnp.float32),
        pltpu.VMEM((NSB, 4 * 1024), jnp.float32),
        pltpu.SemaphoreType.DMA,
        pltpu.SemaphoreType.DMA,
    ],
    compiler_params=pltpu.CompilerParams(
        use_tc_tiling_on_sc=False, needs_layout_passes=False
    ),
)
def _gather_scale(x_hbm, t_hbm, out_hbm, idx_raw, bidx, gbuf, obuf, gsem, ssem):
    wid = lax.axis_index("s") * 2 + lax.axis_index("c")
    lane = lax.iota(jnp.int32, 16)
    lane26 = lane * NUM_COLS

    pltpu.sync_copy(x_hbm.at[pl.ds(wid * PER_W, PER_W)], idx_raw)

    def build_and_issue(n, slot):
        c = lax.shift_right_logical(n, 2)
        rb = lax.bitwise_and(n, 3)

        @plsc.parallel_loop(0, 8, unroll=8)
        def _b(t):
            base = (rb * 128 + t * 16) * NUM_COLS + c
            vals = plsc.load_gather(idx_raw, [lane26 + base])
            bidx[slot, pl.ds(t * 16, 16)] = vals

        pltpu.async_copy(t_hbm.at[bidx.at[slot]], gbuf.at[slot], gsem)

    for p in range(NB - 1):
        build_and_issue(jnp.int32(p), p)

    def step(n, carry):
        slot = lax.rem(n, NB)
        pslot = lax.rem(n + NB - 1, NB)
        sb = lax.rem(n, NSB)
        c = lax.shift_right_logical(n, 2)
        rb = lax.bitwise_and(n, 3)
        rbg = wid * 4 + rb

        @pl.when(n + NB - 1 < NBLK)
        def _lookahead():
            build_and_issue(n + NB - 1, pslot)

        @pl.when(n >= NSB)
        def _wait_store():
            pltpu.make_async_copy(
                obuf.at[sb], out_hbm.at[pl.ds(0, 4 * 1024)], ssem
            ).wait()

        pltpu.make_async_copy(
            t_hbm.at[bidx.at[slot]], gbuf.at[slot], gsem
        ).wait()

        @plsc.parallel_loop(0, EMBED_DIM, unroll=4)
        def _t(f):
            for m in range(8):
                rows = m * 16 + lane
                vals = plsc.load_gather(
                    gbuf,
                    [jnp.full((16,), slot, jnp.int32),
                     rows,
                     jnp.full((16,), f, jnp.int32)],
                )
                obuf[sb, pl.ds(f * 128 + m * 16, 16)] = vals * SCALE

        for fb in range(4):
            base = ((c * 4 + fb) * 128 + rbg) * 1024
            pltpu.async_copy(
                obuf.at[sb, pl.ds(fb * 1024, 1024)],
                out_hbm.at[pl.ds(base, 1024)],
                ssem,
            )
        return carry

    lax.fori_loop(0, NBLK, step, 0)

    for _ in range(NSB):
        pltpu.make_async_copy(
            obuf.at[0], out_hbm.at[pl.ds(0, 4 * 1024)], ssem
        ).wait()


@jax.jit
def kernel(x, table):
    tail = table[VMAIN:].reshape(-1)
    tflat = _transpose_table(table.T, tail)
    tlin = tflat.reshape(VOCAB, EMBED_DIM)
    oflat = _gather_scale(x.reshape(-1), tlin)
    out = oflat.reshape(NUM_COLS, 4, 128, 8, 128)
    return out.transpose(2, 4, 0, 1, 3).reshape(NUM_ROWS, NUM_COLS, EMBED_DIM)
